# baseline (speedup 1.0000x reference)
import functools

import jax
import jax.numpy as jnp
from jax import lax
from jax.experimental import pallas as pl
from jax.experimental.pallas import tpu as pltpu
from jax.experimental.pallas import tpu_sc as plsc

N_NODES = 100000
EMB_DIM = 128
N_EDGES_TOTAL = 600000

NUM_WORKERS = 32
CH = 128
N_PAD = 602112
PER_W = N_PAD // NUM_WORKERS
N_CHUNKS = PER_W // CH


@functools.partial(
    pl.kernel,
    mesh=plsc.VectorSubcoreMesh(core_axis_name="c", subcore_axis_name="s"),
    out_type=jax.ShapeDtypeStruct((N_PAD,), jnp.float32),
    compiler_params=pltpu.CompilerParams(needs_layout_passes=False,
                                         use_tc_tiling_on_sc=False),
    scratch_types=[
        pltpu.VMEM((PER_W,), jnp.int32),
        pltpu.VMEM((PER_W,), jnp.int32),
        pltpu.VMEM((2, CH, EMB_DIM), jnp.float32),
        pltpu.VMEM((2, CH, EMB_DIM), jnp.float32),
        pltpu.VMEM((PER_W,), jnp.float32),
        pltpu.SemaphoreType.DMA,
        pltpu.SemaphoreType.DMA,
    ],
)
def _probe_kernel(table_hbm, src_hbm, dst_hbm, out_hbm,
                  idx_s, idx_d, rows_s, rows_d, out_v, sem0, sem1):
    wid = lax.axis_index("s") * 2 + lax.axis_index("c")
    base_w = wid * PER_W
    sems = (sem0, sem1)

    pltpu.sync_copy(src_hbm.at[pl.ds(base_w, PER_W)], idx_s)
    pltpu.sync_copy(dst_hbm.at[pl.ds(base_w, PER_W)], idx_d)

    def fire(c, buf):
        off = c * CH
        pltpu.async_copy(table_hbm.at[idx_s.at[pl.ds(off, CH)]],
                         rows_s.at[buf], sems[buf])
        pltpu.async_copy(table_hbm.at[idx_d.at[pl.ds(off, CH)]],
                         rows_d.at[buf], sems[buf])

    def drain(buf):
        pltpu.make_async_copy(table_hbm.at[pl.ds(0, CH)],
                              rows_s.at[buf], sems[buf]).wait()
        pltpu.make_async_copy(table_hbm.at[pl.ds(0, CH)],
                              rows_d.at[buf], sems[buf]).wait()

    fire(0, 0)

    def pair_body(p, carry):
        c0 = 2 * p
        fire(c0 + 1, 1)
        drain(0)
        fire(c0 + 2, 0)
        drain(1)
        return carry

    lax.fori_loop(0, (N_CHUNKS - 1) // 2, pair_body, 0)
    drain(0)

    out_v[pl.ds(0, 16)] = rows_s[0, 0, pl.ds(0, 16)]
    pltpu.sync_copy(out_v, out_hbm.at[pl.ds(base_w, PER_W)])


def kernel(x, emb, pos_edge_index, neg_edge_index):
    pad = N_PAD - N_EDGES_TOTAL
    zeros = jnp.zeros((pad,), jnp.int32)
    src = jnp.concatenate([pos_edge_index[0], neg_edge_index[0], zeros])
    dst = jnp.concatenate([pos_edge_index[1], neg_edge_index[1], zeros])
    out = _probe_kernel(emb, src, dst)
    return out[:N_EDGES_TOTAL]


# speedup vs baseline: 1.2114x; 1.2114x over previous
import functools

import jax
import jax.numpy as jnp
from jax import lax
from jax.experimental import pallas as pl
from jax.experimental.pallas import tpu as pltpu
from jax.experimental.pallas import tpu_sc as plsc

N_NODES = 100000
EMB_DIM = 128
N_EDGES_TOTAL = 600000

NUM_WORKERS = 32
CH = 128
N_PAD = 602112
PER_W = N_PAD // NUM_WORKERS
N_CHUNKS = PER_W // CH


@functools.partial(
    pl.kernel,
    mesh=plsc.VectorSubcoreMesh(core_axis_name="c", subcore_axis_name="s"),
    out_type=jax.ShapeDtypeStruct((N_PAD,), jnp.float32),
    compiler_params=pltpu.CompilerParams(needs_layout_passes=False,
                                         use_tc_tiling_on_sc=False),
    scratch_types=[
        pltpu.VMEM((PER_W,), jnp.int32),
        pltpu.VMEM((PER_W,), jnp.int32),
        pltpu.VMEM((3, CH, EMB_DIM), jnp.bfloat16),
        pltpu.VMEM((3, CH, EMB_DIM), jnp.bfloat16),
        pltpu.VMEM((PER_W,), jnp.float32),
        pltpu.VMEM((16 * 17,), jnp.float32),
        pltpu.SemaphoreType.DMA,
        pltpu.SemaphoreType.DMA,
        pltpu.SemaphoreType.DMA,
    ],
)
def _link_logits_kernel(table_hbm, src_hbm, dst_hbm, out_hbm,
                        idx_s, idx_d, rows_s, rows_d, out_v, tr,
                        sem0, sem1, sem2):
    wid = lax.axis_index("s") * 2 + lax.axis_index("c")
    base_w = wid * PER_W
    lane = lax.iota(jnp.int32, 16)
    lane17 = lane * 17
    sems = (sem0, sem1, sem2)

    pltpu.sync_copy(src_hbm.at[pl.ds(base_w, PER_W)], idx_s)
    pltpu.sync_copy(dst_hbm.at[pl.ds(base_w, PER_W)], idx_d)

    def fire(c, buf):
        off = c * CH
        pltpu.async_copy(table_hbm.at[idx_s.at[pl.ds(off, CH)]],
                         rows_s.at[buf], sems[buf])
        pltpu.async_copy(table_hbm.at[idx_d.at[pl.ds(off, CH)]],
                         rows_d.at[buf], sems[buf])

    def drain(buf):
        pltpu.make_async_copy(table_hbm.at[pl.ds(0, CH)],
                              rows_s.at[buf], sems[buf]).wait()
        pltpu.make_async_copy(table_hbm.at[pl.ds(0, CH)],
                              rows_d.at[buf], sems[buf]).wait()

    def compute(c, buf):
        def group_body(g, carry):
            for q in range(4):
                eis = [g * 16 + q * 4 + i for i in range(4)]
                av = [[rows_s[buf, ei, pl.ds(32 * k, 32)] for k in range(4)]
                      for ei in eis]
                bv = [[rows_d[buf, ei, pl.ds(32 * k, 32)] for k in range(4)]
                      for ei in eis]
                prods = [[av[i][k] * bv[i][k] for i in range(4)]
                         for k in range(4)]
                s01 = [prods[0][i] + prods[1][i] for i in range(4)]
                s23 = [prods[2][i] + prods[3][i] for i in range(4)]
                psum = [s01[i] + s23[i] for i in range(4)]
                unp = [plsc.unpack(psum[i], format=plsc.PackFormat.INTERLEAVED)
                       for i in range(4)]
                red = [u0 + u1 for (u0, u1) in unp]
                for i in range(4):
                    tr[pl.ds(17 * (q * 4 + i), 16)] = red[i]
            cols = [plsc.load_gather(tr, [lane17 + j]) for j in range(16)]
            while len(cols) > 1:
                cols = [cols[i] + cols[i + 1] for i in range(0, len(cols), 2)]
            out_v[pl.ds(c * CH + g * 16, 16)] = cols[0]
            return carry

        lax.fori_loop(0, CH // 16, group_body, 0)

    fire(0, 0)
    fire(1, 1)

    def triple_body(t, carry):
        c0 = 3 * t
        for b in range(3):
            fire(c0 + b + 2, (b + 2) % 3)
            drain(b)
            compute(c0 + b, b)
        return carry

    lax.fori_loop(0, N_CHUNKS // 3 - 1, triple_body, 0)

    fire(N_CHUNKS - 1, 2)
    drain(0)
    compute(N_CHUNKS - 3, 0)
    drain(1)
    compute(N_CHUNKS - 2, 1)
    drain(2)
    compute(N_CHUNKS - 1, 2)

    pltpu.sync_copy(out_v, out_hbm.at[pl.ds(base_w, PER_W)])


def kernel(x, emb, pos_edge_index, neg_edge_index):
    emb_bf = emb.astype(jnp.bfloat16)
    pad = N_PAD - N_EDGES_TOTAL
    zeros = jnp.zeros((pad,), jnp.int32)
    src = jnp.concatenate([pos_edge_index[0], neg_edge_index[0], zeros])
    dst = jnp.concatenate([pos_edge_index[1], neg_edge_index[1], zeros])
    out = _link_logits_kernel(emb_bf, src, dst)
    return out[:N_EDGES_TOTAL]
